# Initial kernel scaffold; baseline (speedup 1.0000x reference)
import jax
import jax.numpy as jnp
from jax.experimental import pallas as pl


def kernel(cat_f0, cat_f1, cat_f2, cat_f3, cat_f4, cat_f5, cat_f6, cat_f7, x_price, E0, E1, E2, E3, E4, E5, E6, E7, W1, b1, W2, b2):
    raise NotImplementedError("write your pallas kernel here")


# speedup vs baseline: 4.5884x; 4.5884x over previous
import functools

import jax
import jax.numpy as jnp
from jax import lax
from jax.experimental import pallas as pl
from jax.experimental.pallas import tpu as pltpu
from jax.experimental.pallas import tpu_sc as plsc

_B = 16384
_ED = 128
_NF = 8
_HID = 1024
_CAT = _NF * _ED

_NC = 2
_NS = 16
_NW = _NC * _NS
_BPW = _B // _NW
_CHUNK = 128
_NCHUNK = _BPW // _CHUNK


def _sc_gather(cats, tables):
  mesh = plsc.VectorSubcoreMesh(core_axis_name="c", subcore_axis_name="s")

  @functools.partial(
      pl.kernel,
      out_type=jax.ShapeDtypeStruct((_B, _CAT), jnp.float32),
      mesh=mesh,
      scratch_types=[
          pltpu.VMEM((_NCHUNK, _CHUNK), jnp.int32),
          pltpu.VMEM((_BPW, _ED), jnp.float32),
          pltpu.SemaphoreType.DMA,
      ],
  )
  def gather_kernel(c0, c1, c2, c3, c4, c5, c6, c7,
                    t0, t1, t2, t3, t4, t5, t6, t7,
                    out_hbm, idx_v, rows_v, sem):
    wid = lax.axis_index("s") * _NC + lax.axis_index("c")
    base = wid * _BPW
    row0 = wid * _NCHUNK
    cat_refs = [c0, c1, c2, c3, c4, c5, c6, c7]
    tab_refs = [t0, t1, t2, t3, t4, t5, t6, t7]
    for f in range(_NF):
      pltpu.sync_copy(cat_refs[f].at[pl.ds(row0, _NCHUNK)], idx_v)
      for c in range(_NCHUNK):
        pltpu.async_copy(
            tab_refs[f].at[idx_v.at[c]],
            rows_v.at[pl.ds(c * _CHUNK, _CHUNK)],
            sem,
        )
      for c in range(_NCHUNK):
        pltpu.make_async_copy(
            tab_refs[f].at[idx_v.at[c]],
            rows_v.at[pl.ds(c * _CHUNK, _CHUNK)],
            sem,
        ).wait()
      pltpu.sync_copy(
          rows_v,
          out_hbm.at[pl.ds(base, _BPW), pl.ds(f * _ED, _ED)],
      )

  cats2d = [c.reshape(_B // _CHUNK, _CHUNK) for c in cats]
  return gather_kernel(*cats2d, *tables)


def _mlp_body(emb_ref, price_ref, w1_ref, w1p_ref, b1_ref, w2_ref, b2_ref,
              out_ref):
  h = jnp.dot(emb_ref[...], w1_ref[...], preferred_element_type=jnp.float32)
  h = h + price_ref[...] * w1p_ref[...] + b1_ref[...]
  h = jnp.maximum(h, 0.0)
  o = jnp.dot(h, w2_ref[...], preferred_element_type=jnp.float32)
  o = jnp.maximum(o + b2_ref[...], 0.0)
  out_ref[...] = o


def _mlp(emb, price2d, w1a, w1p, b1, w2, b2):
  nblk = 32
  bb = _B // nblk
  return pl.pallas_call(
      _mlp_body,
      grid=(nblk,),
      in_specs=[
          pl.BlockSpec((bb, _CAT), lambda i: (i, 0)),
          pl.BlockSpec((bb, 1), lambda i: (i, 0)),
          pl.BlockSpec((_CAT, _HID), lambda i: (0, 0)),
          pl.BlockSpec((1, _HID), lambda i: (0, 0)),
          pl.BlockSpec((1, _HID), lambda i: (0, 0)),
          pl.BlockSpec((_HID, _HID), lambda i: (0, 0)),
          pl.BlockSpec((1, _HID), lambda i: (0, 0)),
      ],
      out_specs=pl.BlockSpec((bb, _HID), lambda i: (i, 0)),
      out_shape=jax.ShapeDtypeStruct((_B, _HID), jnp.float32),
      compiler_params=pltpu.CompilerParams(
          dimension_semantics=("arbitrary",),
      ),
  )(emb, price2d, w1a, w1p, b1, w2, b2)


def kernel(cat_f0, cat_f1, cat_f2, cat_f3, cat_f4, cat_f5, cat_f6, cat_f7,
           x_price, E0, E1, E2, E3, E4, E5, E6, E7, W1, b1, W2, b2):
  cats = [cat_f0, cat_f1, cat_f2, cat_f3, cat_f4, cat_f5, cat_f6, cat_f7]
  tables = [E0, E1, E2, E3, E4, E5, E6, E7]
  emb = _sc_gather(cats, tables)
  w1a = W1[:_CAT]
  w1p = W1[_CAT:]
  return _mlp(emb, x_price[:, None], w1a, w1p, b1[None, :], W2, b2[None, :])


# speedup vs baseline: 4.6215x; 1.0072x over previous
import functools

import jax
import jax.numpy as jnp
from jax import lax
from jax.experimental import pallas as pl
from jax.experimental.pallas import tpu as pltpu
from jax.experimental.pallas import tpu_sc as plsc

_B = 16384
_ED = 128
_NF = 8
_HID = 1024
_CAT = _NF * _ED

_NC = 2
_NS = 16
_NW = _NC * _NS
_BPW = _B // _NW
_CHUNK = 128
_NCHUNK = _BPW // _CHUNK


def _sc_gather(cats, tables):
  mesh = plsc.VectorSubcoreMesh(core_axis_name="c", subcore_axis_name="s")

  @functools.partial(
      pl.kernel,
      out_type=jax.ShapeDtypeStruct((_B, _CAT), jnp.float32),
      mesh=mesh,
      scratch_types=[
          pltpu.VMEM((_NCHUNK, _CHUNK), jnp.int32),
          pltpu.VMEM((_BPW, _ED), jnp.float32),
          pltpu.SemaphoreType.DMA,
      ],
  )
  def gather_kernel(c0, c1, c2, c3, c4, c5, c6, c7,
                    t0, t1, t2, t3, t4, t5, t6, t7,
                    out_hbm, idx_v, rows_v, sem):
    wid = lax.axis_index("s") * _NC + lax.axis_index("c")
    base = wid * _BPW
    row0 = wid * _NCHUNK
    cat_refs = [c0, c1, c2, c3, c4, c5, c6, c7]
    tab_refs = [t0, t1, t2, t3, t4, t5, t6, t7]
    for f in range(_NF):
      pltpu.sync_copy(cat_refs[f].at[pl.ds(row0, _NCHUNK)], idx_v)
      for c in range(_NCHUNK):
        pltpu.async_copy(
            tab_refs[f].at[idx_v.at[c]],
            rows_v.at[pl.ds(c * _CHUNK, _CHUNK)],
            sem,
        )
      for c in range(_NCHUNK):
        pltpu.make_async_copy(
            tab_refs[f].at[idx_v.at[c]],
            rows_v.at[pl.ds(c * _CHUNK, _CHUNK)],
            sem,
        ).wait()
      pltpu.sync_copy(
          rows_v,
          out_hbm.at[pl.ds(base, _BPW), pl.ds(f * _ED, _ED)],
      )

  cats2d = [c.reshape(_B // _CHUNK, _CHUNK) for c in cats]
  return gather_kernel(*cats2d, *tables)


def _mlp_body(emb_ref, price_ref, w1_ref, w1p_ref, b1_ref, w2_ref, b2_ref,
              out_ref):
  h = jnp.dot(emb_ref[...].astype(jnp.bfloat16), w1_ref[...],
              preferred_element_type=jnp.float32)
  h = h + price_ref[...] * w1p_ref[...] + b1_ref[...]
  h = jnp.maximum(h, 0.0)
  o = jnp.dot(h.astype(jnp.bfloat16), w2_ref[...],
              preferred_element_type=jnp.float32)
  o = jnp.maximum(o + b2_ref[...], 0.0)
  out_ref[...] = o


def _mlp(emb, price2d, w1a, w1p, b1, w2, b2):
  nblk = 32
  bb = _B // nblk
  return pl.pallas_call(
      _mlp_body,
      grid=(nblk,),
      in_specs=[
          pl.BlockSpec((bb, _CAT), lambda i: (i, 0)),
          pl.BlockSpec((bb, 1), lambda i: (i, 0)),
          pl.BlockSpec((_CAT, _HID), lambda i: (0, 0)),
          pl.BlockSpec((1, _HID), lambda i: (0, 0)),
          pl.BlockSpec((1, _HID), lambda i: (0, 0)),
          pl.BlockSpec((_HID, _HID), lambda i: (0, 0)),
          pl.BlockSpec((1, _HID), lambda i: (0, 0)),
      ],
      out_specs=pl.BlockSpec((bb, _HID), lambda i: (i, 0)),
      out_shape=jax.ShapeDtypeStruct((_B, _HID), jnp.float32),
      compiler_params=pltpu.CompilerParams(
          dimension_semantics=("arbitrary",),
      ),
  )(emb, price2d, w1a, w1p, b1, w2, b2)


def kernel(cat_f0, cat_f1, cat_f2, cat_f3, cat_f4, cat_f5, cat_f6, cat_f7,
           x_price, E0, E1, E2, E3, E4, E5, E6, E7, W1, b1, W2, b2):
  cats = [cat_f0, cat_f1, cat_f2, cat_f3, cat_f4, cat_f5, cat_f6, cat_f7]
  tables = [E0, E1, E2, E3, E4, E5, E6, E7]
  emb = _sc_gather(cats, tables)
  w1a = W1[:_CAT].astype(jnp.bfloat16)
  w1p = W1[_CAT:]
  return _mlp(emb, x_price[:, None], w1a, w1p, b1[None, :],
              W2.astype(jnp.bfloat16), b2[None, :])
